# baseline (speedup 1.0000x reference)
import functools

import jax
import jax.numpy as jnp
from jax import lax
from jax.experimental import pallas as pl
from jax.experimental.pallas import tpu as pltpu
from jax.experimental.pallas import tpu_sc as plsc

B = 16384
D = 32
K = 20
NC = 2
NS = 16
NW = NC * NS
BPW = B // NW
CH = 128
NCHUNK = BPW // CH
NGRP = CH // 16


def _sc_body(cidx_hbm, xidx_hbm, nidx_hbm, in_emb, out_emb,
             ctx_out, neg_out,
             cidx_v, xidx_v, nidx_v, crows, xrows, nrows,
             ctx_v, neg_v, sem):
    wid = lax.axis_index("s") * NC + lax.axis_index("c")
    base = wid * BPW

    pltpu.sync_copy(cidx_hbm.at[pl.ds(wid * (BPW // 128), BPW // 128)], cidx_v)
    pltpu.sync_copy(xidx_hbm.at[pl.ds(wid * (BPW // 128), BPW // 128)], xidx_v)
    nrows_pw = BPW * K // 128
    pltpu.sync_copy(nidx_hbm.at[pl.ds(wid * nrows_pw, nrows_pw)], nidx_v)

    iota16 = lax.iota(jnp.int32, 16)

    def chunk_body(c, _):
        cp_c = pltpu.async_copy(in_emb.at[cidx_v.at[c]], crows, sem)
        cp_x = pltpu.async_copy(out_emb.at[xidx_v.at[c]], xrows, sem)
        cps = []
        for j in range(K):
            cps.append(pltpu.async_copy(
                out_emb.at[nidx_v.at[c * K + j]],
                nrows.at[pl.ds(j * CH, CH)], sem))
        cp_c.wait()
        cp_x.wait()
        for cp in cps:
            cp.wait()

        def group_body(g, _):
            row = g * 16 + iota16
            out_off = c * CH + g * 16
            c_cols = [plsc.load_gather(crows, [row, jnp.full((16,), d, jnp.int32)])
                      for d in range(D)]
            acc = c_cols[0] * plsc.load_gather(
                xrows, [row, jnp.zeros((16,), jnp.int32)])
            for d in range(1, D):
                acc = acc + c_cols[d] * plsc.load_gather(
                    xrows, [row, jnp.full((16,), d, jnp.int32)])
            ctx_v[pl.ds(out_off, 16)] = acc

            r20 = row * K

            def k_body(k, _):
                rk = r20 + k
                nacc = c_cols[0] * plsc.load_gather(
                    nrows, [rk, jnp.zeros((16,), jnp.int32)])
                for d in range(1, D):
                    nacc = nacc + c_cols[d] * plsc.load_gather(
                        nrows, [rk, jnp.full((16,), d, jnp.int32)])
                neg_v[k, pl.ds(out_off, 16)] = nacc
                return 0

            lax.fori_loop(0, K, k_body, 0)
            return 0

        lax.fori_loop(0, NGRP, group_body, 0)
        return 0

    lax.fori_loop(0, NCHUNK, chunk_body, 0)

    pltpu.sync_copy(ctx_v, ctx_out.at[pl.ds(base, BPW)])
    pltpu.sync_copy(neg_v, neg_out.at[:, pl.ds(base, BPW)])


_sc_affinities = functools.partial(
    pl.kernel,
    out_type=(
        jax.ShapeDtypeStruct((B,), jnp.float32),
        jax.ShapeDtypeStruct((K, B), jnp.float32),
    ),
    mesh=plsc.VectorSubcoreMesh(core_axis_name="c", subcore_axis_name="s"),
    compiler_params=pltpu.CompilerParams(
        needs_layout_passes=False, use_tc_tiling_on_sc=False),
    scratch_types=(
        pltpu.VMEM((BPW // 128, 128), jnp.int32),
        pltpu.VMEM((BPW // 128, 128), jnp.int32),
        pltpu.VMEM((BPW * K // 128, 128), jnp.int32),
        pltpu.VMEM((CH, D), jnp.float32),
        pltpu.VMEM((CH, D), jnp.float32),
        pltpu.VMEM((CH * K, D), jnp.float32),
        pltpu.VMEM((BPW,), jnp.float32),
        pltpu.VMEM((K, BPW), jnp.float32),
        pltpu.SemaphoreType.DMA,
    ),
)(_sc_body)


def _loss_body(ctx_ref, neg_ref, out_ref):
    ctx = ctx_ref[...]
    neg = -neg_ref[...]
    ls_c = jnp.minimum(ctx, 0.0) - jnp.log1p(jnp.exp(-jnp.abs(ctx)))
    ls_n = jnp.minimum(neg, 0.0) - jnp.log1p(jnp.exp(-jnp.abs(neg)))
    out_ref[0, 0] = -(jnp.sum(ls_c) / B) - (jnp.sum(ls_n) / (B * K))


def kernel(center, context, negatives, input_embedding, output_embedding):
    cidx = center.astype(jnp.int32).reshape(B // 128, 128)
    xidx = context.astype(jnp.int32).reshape(B // 128, 128)
    nidx = negatives.astype(jnp.int32).reshape(B * K // 128, 128)

    ctx_aff, neg_aff = _sc_affinities(
        cidx, xidx, nidx, input_embedding, output_embedding)

    loss = pl.pallas_call(
        _loss_body,
        out_shape=jax.ShapeDtypeStruct((1, 1), jnp.float32),
        out_specs=pl.BlockSpec(memory_space=pltpu.SMEM),
    )(ctx_aff.reshape(B // 128, 128), neg_aff.reshape(K * B // 128, 128))
    return loss[0, 0]
